# Initial kernel scaffold; baseline (speedup 1.0000x reference)
import jax
import jax.numpy as jnp
from jax.experimental import pallas as pl


def kernel(coords, node_feat, edge_attr, params, edge_index, batch_index, t):
    raise NotImplementedError("write your pallas kernel here")


# speedup vs baseline: 1.0229x; 1.0229x over previous
import functools

import jax
import jax.numpy as jnp
from jax.experimental import pallas as pl
from jax.experimental.pallas import tpu as pltpu

_SLOPE = 0.01


def _leaky(v):
    return jnp.where(v > 0, v, _SLOPE * v)


def _edge_tile_body(H, z_ref, e_ref, w2_ref, b2_ref, w1c_ref, w1e_ref,
                    cw_ref, cb_ref, eow_ref, eob_ref,
                    m_ref, t_ref, en_ref):
    z = z_ref[...]
    z0 = z[:, :H]
    dx = z[:, H:H + 1]
    dy = z[:, H + 1:H + 2]
    d2 = dx * dx + dy * dy
    e = e_ref[...]
    z1 = (z0 + d2 * w1c_ref[...]
          + jnp.dot(e, w1e_ref[...], preferred_element_type=jnp.float32))
    m1 = _leaky(z1)
    m = _leaky(jnp.dot(m1, w2_ref[...], preferred_element_type=jnp.float32)
               + b2_ref[...])
    m_ref[...] = m
    coef = jnp.dot(m, cw_ref[...], preferred_element_type=jnp.float32) + cb_ref[...]
    s = coef / (jnp.sqrt(d2) + 1.0)
    t_ref[...] = jnp.concatenate([dx * s, dy * s], axis=1)
    en_ref[...] = (jnp.dot(m, eow_ref[...], preferred_element_type=jnp.float32)
                   + eob_ref[...])


def _edge_mlp(Z, e, p, H, EF, TE):
    E = Z.shape[0]
    Hp = Z.shape[1]
    grid = E // TE
    w2 = p['edge_w2']
    b2 = p['edge_b2'].reshape(1, H)
    w1c = p['edge_w1'][2 * H].reshape(1, H)
    w1e = p['edge_w1'][2 * H + 1:]
    cw = p['coord_w']
    cb = p['coord_b'].reshape(1, 1)
    eow = p['edge_out_w']
    eob = p['edge_out_b'].reshape(1, EF)
    full = lambda shape: pl.BlockSpec(shape, lambda i: (0, 0))
    return pl.pallas_call(
        functools.partial(_edge_tile_body, H),
        grid=(grid,),
        in_specs=[
            pl.BlockSpec((TE, Hp), lambda i: (i, 0)),
            pl.BlockSpec((TE, EF), lambda i: (i, 0)),
            full(w2.shape), full(b2.shape), full(w1c.shape), full(w1e.shape),
            full(cw.shape), full(cb.shape), full(eow.shape), full(eob.shape),
        ],
        out_specs=[
            pl.BlockSpec((TE, H), lambda i: (i, 0)),
            pl.BlockSpec((TE, 2), lambda i: (i, 0)),
            pl.BlockSpec((TE, EF), lambda i: (i, 0)),
        ],
        out_shape=[
            jax.ShapeDtypeStruct((E, H), jnp.float32),
            jax.ShapeDtypeStruct((E, 2), jnp.float32),
            jax.ShapeDtypeStruct((E, EF), jnp.float32),
        ],
    )(Z, e, w2, b2, w1c, w1e, cw, cb, eow, eob)


def _layer(p, h, x, e, src, dst, deg, H, EF, TE):
    n = h.shape[0]
    P = h @ p['edge_w1'][:H] + p['edge_b1']
    Q = h @ p['edge_w1'][H:2 * H]
    pad = jnp.zeros((n, 14), dtype=jnp.float32)
    Px = jnp.concatenate([P, x, pad], axis=1)
    Qx = jnp.concatenate([Q, -x, pad], axis=1)
    Z = jnp.take(Px, dst, axis=0) + jnp.take(Qx, src, axis=0)
    m, trans, e_new = _edge_mlp(Z, e, p, H, EF, TE)
    msum = jax.ops.segment_sum(m, dst, num_segments=n)
    tsum = jax.ops.segment_sum(trans, dst, num_segments=n)
    x_new = x + tsum / deg[:, None]
    m_agg = msum / deg[:, None]
    hn = _leaky(h @ p['node_w1'][:H] + m_agg @ p['node_w1'][H:] + p['node_b1'])
    hn = hn @ p['node_w2'] + p['node_b2']
    mean = jnp.mean(hn, axis=0)
    var = jnp.var(hn, axis=0)
    hn = (hn - mean) / jnp.sqrt(var + 1e-5) * p['bn_gamma'] + p['bn_beta']
    hn = _leaky(hn)
    return hn, x_new, e_new


def kernel(coords, node_feat, edge_attr, params, edge_index, batch_index, t):
    n, d = node_feat.shape
    E = edge_attr.shape[0]
    EF = edge_attr.shape[1]
    H = 2 * d
    TE = 2560 if E % 2560 == 0 else E
    t_idx = jnp.full((n,), t, dtype=jnp.int32)
    t_embed = jnp.take(params['time_embed'], t_idx, axis=0)
    h = jnp.concatenate([node_feat, t_embed], axis=-1)
    x = coords
    e = edge_attr
    src = edge_index[0]
    dst = edge_index[1]
    deg = jax.ops.segment_sum(jnp.ones((E,), dtype=jnp.float32), dst,
                              num_segments=n)
    deg = jnp.maximum(deg, 1.0)
    for p in params['layers']:
        h, x, e = _layer(p, h, x, e, src, dst, deg, H, EF, TE)
    pred_noise = h @ params['readout_w'] + params['readout_b']
    return (x, pred_noise)


# speedup vs baseline: 3.4201x; 3.3434x over previous
import functools

import jax
import jax.numpy as jnp
from jax import lax
from jax.experimental import pallas as pl
from jax.experimental.pallas import tpu as pltpu
from jax.experimental.pallas import tpu_sc as plsc

_SLOPE = 0.01

_NW = 32
_K = 80


def _leaky(v):
    return jnp.where(v > 0, v, _SLOPE * v)


def _gather_pq(Px, Qx, x_flat, src, dst):
    N, Wp = Px.shape
    E = src.shape[0]
    per_w = E // _NW
    steps = per_w // _K
    mesh = plsc.VectorSubcoreMesh(core_axis_name="c", subcore_axis_name="s")

    @functools.partial(
        pl.kernel, mesh=mesh,
        out_type=[jax.ShapeDtypeStruct((E, Wp), jnp.float32),
                  jax.ShapeDtypeStruct((2 * E,), jnp.float32)],
        scratch_types=[
            pltpu.VMEM((per_w,), jnp.int32),
            pltpu.VMEM((per_w,), jnp.int32),
            pltpu.VMEM((_K, Wp), jnp.float32),
            pltpu.VMEM((_K, Wp), jnp.float32),
            pltpu.VMEM((_K, Wp), jnp.float32),
            pltpu.VMEM((_K, Wp), jnp.float32),
            pltpu.VMEM((2 * _K,), jnp.float32),
            pltpu.VMEM((2 * _K,), jnp.float32),
            pltpu.VMEM((x_flat.shape[0],), jnp.float32),
            pltpu.SemaphoreType.DMA,
            pltpu.SemaphoreType.DMA,
        ],
        compiler_params=pltpu.CompilerParams(needs_layout_passes=False),
    )
    def gather_kernel(px_hbm, qx_hbm, xf_hbm, src_hbm, dst_hbm,
                      zs_hbm, df_hbm,
                      di_all, si_all, rp0, rq0, rp1, rq1, db0, db1, xv,
                      gsem0, gsem1):
        wid = lax.axis_index("s") * 2 + lax.axis_index("c")
        w_base = wid * per_w
        pltpu.sync_copy(xf_hbm, xv)
        pltpu.sync_copy(dst_hbm.at[pl.ds(w_base, per_w)], di_all)
        pltpu.sync_copy(src_hbm.at[pl.ds(w_base, per_w)], si_all)
        lane = lax.iota(jnp.int32, 16)
        bufs = ((rp0, rq0, db0, gsem0), (rp1, rq1, db1, gsem1))

        def issue(b, c):
            rp, rq, _, gsem = bufs[b]
            off = c * _K
            pltpu.async_copy(px_hbm.at[di_all.at[pl.ds(off, _K)]], rp, gsem)
            pltpu.async_copy(qx_hbm.at[si_all.at[pl.ds(off, _K)]], rq, gsem)

        def proc(b, c):
            rp, rq, db, gsem = bufs[b]
            off = c * _K
            pltpu.make_async_copy(
                px_hbm.at[di_all.at[pl.ds(off, _K)]], rp, gsem).wait()
            pltpu.make_async_copy(
                qx_hbm.at[si_all.at[pl.ds(off, _K)]], rq, gsem).wait()

            def addrow(r, _):
                for t in range(Wp // 16):
                    sl = pl.ds(16 * t, 16)
                    rp[r, sl] = rp[r, sl] + rq[r, sl]
                return ()

            lax.fori_loop(0, _K, addrow, ())
            for j in range(_K // 16):
                d16 = di_all[pl.ds(off + 16 * j, 16)]
                s16 = si_all[pl.ds(off + 16 * j, 16)]
                dxv = (plsc.load_gather(xv, [d16 * 2])
                       - plsc.load_gather(xv, [s16 * 2]))
                dyv = (plsc.load_gather(xv, [d16 * 2 + 1])
                       - plsc.load_gather(xv, [s16 * 2 + 1]))
                pos = 32 * j + 2 * lane
                plsc.store_scatter(db, [pos], dxv)
                plsc.store_scatter(db, [pos + 1], dyv)
            base = w_base + off
            pltpu.sync_copy(rp, zs_hbm.at[pl.ds(base, _K)])
            pltpu.sync_copy(db, df_hbm.at[pl.ds(2 * base, 2 * _K)])

        issue(0, 0)

        def body(j, _):
            c = 2 * j
            issue(1, c + 1)
            proc(0, c)

            @pl.when(c + 2 < steps)
            def _():
                issue(0, c + 2)

            proc(1, c + 1)
            return ()

        lax.fori_loop(0, steps // 2, body, ())
        proc(0, steps - 1)

    return gather_kernel(Px, Qx, x_flat, src, dst)


def _scatter_mt(mt, dst):
    E, W = mt.shape
    n_rows = 10240
    stripe = n_rows // 16
    CH = 32
    mesh = plsc.VectorSubcoreMesh(core_axis_name="c", subcore_axis_name="s")

    @functools.partial(
        pl.kernel, mesh=mesh,
        out_type=jax.ShapeDtypeStruct((4, n_rows, 128), jnp.float32),
        scratch_types=[
            pltpu.VMEM((_K,), jnp.int32),
            pltpu.VMEM((_K,), jnp.int32),
            pltpu.VMEM((_K, 128), jnp.float32),
            pltpu.VMEM((_K, 128), jnp.float32),
            pltpu.VMEM((CH, 128), jnp.float32),
            pltpu.VMEM_SHARED((n_rows, 128), jnp.float32),
            pltpu.SemaphoreType.DMA,
            pltpu.SemaphoreType.DMA,
        ],
        compiler_params=pltpu.CompilerParams(needs_layout_passes=False),
    )
    def scatter_kernel(mt_hbm, dst_hbm, out_hbm,
                       di0, di1, rows0, rows1, zb_v, acc, sem0, sem1):
        c = lax.axis_index("c")
        s = lax.axis_index("s")
        bufs = ((di0, rows0, sem0), (di1, rows1, sem1))

        def zero_acc():
            def zrow(r, _):
                for j in range(128 // 16):
                    zb_v[r, pl.ds(16 * j, 16)] = jnp.zeros((16,), jnp.float32)
                return ()
            lax.fori_loop(0, CH, zrow, ())
            for k in range(stripe // CH):
                pltpu.sync_copy(zb_v, acc.at[pl.ds(s * stripe + k * CH, CH)])
            plsc.subcore_barrier()

        def scan(c0, e_base, tile_e):
            n_chunks = tile_e // _K

            def issue(b, i):
                di, rows, sem = bufs[b]
                base = e_base + s * tile_e + i * _K
                pltpu.async_copy(dst_hbm.at[pl.ds(base, _K)], di, sem)
                pltpu.async_copy(
                    mt_hbm.at[pl.ds(base, _K), pl.ds(c0, 128)], rows, sem)

            def proc(b, i):
                di, rows, sem = bufs[b]
                base = e_base + s * tile_e + i * _K
                pltpu.make_async_copy(
                    dst_hbm.at[pl.ds(base, _K)], di, sem).wait()
                pltpu.make_async_copy(
                    mt_hbm.at[pl.ds(base, _K), pl.ds(c0, 128)], rows,
                    sem).wait()
                pltpu.sync_copy(rows, acc.at[di], add=True)

            issue(0, 0)

            def body(j, _):
                i = 2 * j
                issue(1, i + 1)
                proc(0, i)

                @pl.when(i + 2 < n_chunks)
                def _():
                    issue(0, i + 2)

                proc(1, i + 1)
                return ()

            lax.fori_loop(0, n_chunks // 2, body, ())
            if n_chunks % 2:
                proc(0, n_chunks - 1)
            plsc.subcore_barrier()

        def dump(out_idx):
            for k in range(stripe // CH):
                pltpu.sync_copy(acc.at[pl.ds(s * stripe + k * CH, CH)], zb_v)
                pltpu.sync_copy(
                    zb_v, out_hbm.at[out_idx, pl.ds(s * stripe + k * CH, CH)])

        zero_acc()

        @pl.when(c == 0)
        def _():
            scan(0, 0, E // 16)
            dump(0)

        @pl.when(c == 1)
        def _():
            scan(128, 0, E // 16)
            dump(1)

        plsc.subcore_barrier()
        zero_acc()

        @pl.when(c == 0)
        def _():
            scan(256, 0, E // 32)
            dump(2)

        @pl.when(c == 1)
        def _():
            scan(256, E // 2, E // 32)
            dump(3)

    return scatter_kernel(mt, dst)


def _edge_tile_body(H, zs_ref, d_ref, e_ref, w2_ref, b2_ref, w1c_ref,
                    w1e_ref, cw_ref, cb_ref, eow_ref, eob_ref,
                    mt_ref, en_ref):
    TE = zs_ref.shape[0]
    z0 = zs_ref[...]
    dx = d_ref[:, 0:1]
    dy = d_ref[:, 1:2]
    d2 = dx * dx + dy * dy
    e = e_ref[...]
    z1 = (z0 + d2 * w1c_ref[...]
          + jnp.dot(e, w1e_ref[...], preferred_element_type=jnp.float32))
    m1 = _leaky(z1)
    m = _leaky(jnp.dot(m1, w2_ref[...], preferred_element_type=jnp.float32)
               + b2_ref[...])
    mt_ref[:, :H] = m
    coef = jnp.dot(m, cw_ref[...], preferred_element_type=jnp.float32) + cb_ref[...]
    s = coef / (jnp.sqrt(d2) + 1.0)
    col = lax.broadcasted_iota(jnp.int32, (TE, 128), 1)
    t128 = jnp.where(col == 0, dx * s,
                     jnp.where(col == 1, dy * s,
                               jnp.where(col == 2, 1.0, 0.0)))
    mt_ref[:, H:H + 128] = t128
    en_ref[...] = (jnp.dot(m, eow_ref[...], preferred_element_type=jnp.float32)
                   + eob_ref[...])


def _edge_mlp(Zs, diff, e, p, H, EF, TE):
    E = Zs.shape[0]
    grid = E // TE
    w2 = p['edge_w2']
    b2 = p['edge_b2'].reshape(1, H)
    w1c = p['edge_w1'][2 * H].reshape(1, H)
    w1e = p['edge_w1'][2 * H + 1:]
    cw = p['coord_w']
    cb = p['coord_b'].reshape(1, 1)
    eow = p['edge_out_w']
    eob = p['edge_out_b'].reshape(1, EF)
    full = lambda shape: pl.BlockSpec(shape, lambda i: (0, 0))
    return pl.pallas_call(
        functools.partial(_edge_tile_body, H),
        grid=(grid,),
        in_specs=[
            pl.BlockSpec((TE, H), lambda i: (i, 0)),
            pl.BlockSpec((TE, 2), lambda i: (i, 0)),
            pl.BlockSpec((TE, EF), lambda i: (i, 0)),
            full(w2.shape), full(b2.shape), full(w1c.shape), full(w1e.shape),
            full(cw.shape), full(cb.shape), full(eow.shape), full(eob.shape),
        ],
        out_specs=[
            pl.BlockSpec((TE, H + 128), lambda i: (i, 0)),
            pl.BlockSpec((TE, EF), lambda i: (i, 0)),
        ],
        out_shape=[
            jax.ShapeDtypeStruct((E, H + 128), jnp.float32),
            jax.ShapeDtypeStruct((E, EF), jnp.float32),
        ],
    )(Zs, diff, e, w2, b2, w1c, w1e, cw, cb, eow, eob)


def _layer(p, h, x, e, src, dst, H, EF, TE):
    n = h.shape[0]
    P = h @ p['edge_w1'][:H] + p['edge_b1']
    Q = h @ p['edge_w1'][H:2 * H]
    Zs, dflat = _gather_pq(P, Q, x.reshape(-1), src, dst)
    diff = dflat.reshape(-1, 2)
    mt, e_new = _edge_mlp(Zs, diff, e, p, H, EF, TE)
    agg = _scatter_mt(mt, dst)
    msum = jnp.concatenate([agg[0, :n], agg[1, :n]], axis=1)
    tpart = agg[2, :n, :3] + agg[3, :n, :3]
    tsum = tpart[:, :2]
    deg = jnp.maximum(tpart[:, 2], 1.0)
    x_new = x + tsum / deg[:, None]
    m_agg = msum / deg[:, None]
    hn = _leaky(h @ p['node_w1'][:H] + m_agg @ p['node_w1'][H:] + p['node_b1'])
    hn = hn @ p['node_w2'] + p['node_b2']
    mean = jnp.mean(hn, axis=0)
    var = jnp.var(hn, axis=0)
    hn = (hn - mean) / jnp.sqrt(var + 1e-5) * p['bn_gamma'] + p['bn_beta']
    hn = _leaky(hn)
    return hn, x_new, e_new


def kernel(coords, node_feat, edge_attr, params, edge_index, batch_index, t):
    n, d = node_feat.shape
    E = edge_attr.shape[0]
    EF = edge_attr.shape[1]
    H = 2 * d
    TE = 2560 if E % 2560 == 0 else E
    t_idx = jnp.full((n,), t, dtype=jnp.int32)
    t_embed = jnp.take(params['time_embed'], t_idx, axis=0)
    h = jnp.concatenate([node_feat, t_embed], axis=-1)
    x = coords
    e = edge_attr
    src = edge_index[0]
    dst = edge_index[1]
    for p in params['layers']:
        h, x, e = _layer(p, h, x, e, src, dst, H, EF, TE)
    pred_noise = h @ params['readout_w'] + params['readout_b']
    return (x, pred_noise)


# speedup vs baseline: 3.4600x; 1.0116x over previous
import functools

import jax
import jax.numpy as jnp
from jax import lax
from jax.experimental import pallas as pl
from jax.experimental.pallas import tpu as pltpu
from jax.experimental.pallas import tpu_sc as plsc

_SLOPE = 0.01

_NW = 32
_K = 80


def _leaky(v):
    return jnp.where(v > 0, v, _SLOPE * v)


def _gather_pq(Px, Qx, x_flat, src, dst):
    N, Wp = Px.shape
    E = src.shape[0]
    per_w = E // _NW
    steps = per_w // _K
    mesh = plsc.VectorSubcoreMesh(core_axis_name="c", subcore_axis_name="s")

    @functools.partial(
        pl.kernel, mesh=mesh,
        out_type=[jax.ShapeDtypeStruct((E, Wp), jnp.float32),
                  jax.ShapeDtypeStruct((2 * E,), jnp.float32)],
        scratch_types=[
            pltpu.VMEM((per_w,), jnp.int32),
            pltpu.VMEM((per_w,), jnp.int32),
            pltpu.VMEM((_K, Wp), jnp.float32),
            pltpu.VMEM((_K, Wp), jnp.float32),
            pltpu.VMEM((_K, Wp), jnp.float32),
            pltpu.VMEM((_K, Wp), jnp.float32),
            pltpu.VMEM((2 * _K,), jnp.float32),
            pltpu.VMEM((2 * _K,), jnp.float32),
            pltpu.VMEM((x_flat.shape[0],), jnp.float32),
            pltpu.SemaphoreType.DMA,
            pltpu.SemaphoreType.DMA,
        ],
        compiler_params=pltpu.CompilerParams(needs_layout_passes=False),
    )
    def gather_kernel(px_hbm, qx_hbm, xf_hbm, src_hbm, dst_hbm,
                      zs_hbm, df_hbm,
                      di_all, si_all, rp0, rq0, rp1, rq1, db0, db1, xv,
                      gsem0, gsem1):
        wid = lax.axis_index("s") * 2 + lax.axis_index("c")
        w_base = wid * per_w
        pltpu.sync_copy(xf_hbm, xv)
        pltpu.sync_copy(dst_hbm.at[pl.ds(w_base, per_w)], di_all)
        pltpu.sync_copy(src_hbm.at[pl.ds(w_base, per_w)], si_all)
        lane = lax.iota(jnp.int32, 16)
        bufs = ((rp0, rq0, db0, gsem0), (rp1, rq1, db1, gsem1))

        def issue(b, c):
            rp, rq, _, gsem = bufs[b]
            off = c * _K
            pltpu.async_copy(px_hbm.at[di_all.at[pl.ds(off, _K)]], rp, gsem)
            pltpu.async_copy(qx_hbm.at[si_all.at[pl.ds(off, _K)]], rq, gsem)

        def proc(b, c):
            rp, rq, db, gsem = bufs[b]
            off = c * _K
            pltpu.make_async_copy(
                px_hbm.at[di_all.at[pl.ds(off, _K)]], rp, gsem).wait()
            pltpu.make_async_copy(
                qx_hbm.at[si_all.at[pl.ds(off, _K)]], rq, gsem).wait()

            def addrow(r, _):
                for t in range(Wp // 16):
                    sl = pl.ds(16 * t, 16)
                    rp[r, sl] = rp[r, sl] + rq[r, sl]
                return ()

            lax.fori_loop(0, _K, addrow, ())
            for j in range(_K // 16):
                d16 = di_all[pl.ds(off + 16 * j, 16)]
                s16 = si_all[pl.ds(off + 16 * j, 16)]
                dxv = (plsc.load_gather(xv, [d16 * 2])
                       - plsc.load_gather(xv, [s16 * 2]))
                dyv = (plsc.load_gather(xv, [d16 * 2 + 1])
                       - plsc.load_gather(xv, [s16 * 2 + 1]))
                pos = 32 * j + 2 * lane
                plsc.store_scatter(db, [pos], dxv)
                plsc.store_scatter(db, [pos + 1], dyv)
            base = w_base + off
            pltpu.sync_copy(rp, zs_hbm.at[pl.ds(base, _K)])
            pltpu.sync_copy(db, df_hbm.at[pl.ds(2 * base, 2 * _K)])

        issue(0, 0)

        def body(j, _):
            c = 2 * j
            issue(1, c + 1)
            proc(0, c)

            @pl.when(c + 2 < steps)
            def _():
                issue(0, c + 2)

            proc(1, c + 1)
            return ()

        lax.fori_loop(0, steps // 2, body, ())
        proc(0, steps - 1)

    return gather_kernel(Px, Qx, x_flat, src, dst)


def _scatter_mt(mt, dst):
    E, W = mt.shape
    n_rows = 10240
    stripe = n_rows // 16
    CH = 32
    mesh = plsc.VectorSubcoreMesh(core_axis_name="c", subcore_axis_name="s")

    @functools.partial(
        pl.kernel, mesh=mesh,
        out_type=jax.ShapeDtypeStruct((4, n_rows, 128), jnp.float32),
        scratch_types=[
            pltpu.VMEM((_K,), jnp.int32),
            pltpu.VMEM((_K,), jnp.int32),
            pltpu.VMEM((_K, 128), jnp.float32),
            pltpu.VMEM((_K, 128), jnp.float32),
            pltpu.VMEM((CH, 128), jnp.float32),
            pltpu.VMEM_SHARED((n_rows, 128), jnp.float32),
            pltpu.SemaphoreType.DMA,
            pltpu.SemaphoreType.DMA,
        ],
        compiler_params=pltpu.CompilerParams(needs_layout_passes=False),
    )
    def scatter_kernel(mt_hbm, dst_hbm, out_hbm,
                       di0, di1, rows0, rows1, zb_v, acc, sem0, sem1):
        c = lax.axis_index("c")
        s = lax.axis_index("s")
        bufs = ((di0, rows0, sem0), (di1, rows1, sem1))

        def zero_acc():
            def zrow(r, _):
                for j in range(128 // 16):
                    zb_v[r, pl.ds(16 * j, 16)] = jnp.zeros((16,), jnp.float32)
                return ()
            lax.fori_loop(0, CH, zrow, ())
            for k in range(stripe // CH):
                pltpu.sync_copy(zb_v, acc.at[pl.ds(s * stripe + k * CH, CH)])
            plsc.subcore_barrier()

        def scan(c0, e_base, tile_e):
            n_chunks = tile_e // _K

            def issue(b, i):
                di, rows, sem = bufs[b]
                base = e_base + s * tile_e + i * _K
                pltpu.async_copy(dst_hbm.at[pl.ds(base, _K)], di, sem)
                pltpu.async_copy(
                    mt_hbm.at[pl.ds(base, _K), pl.ds(c0, 128)], rows, sem)

            def proc(b, i):
                di, rows, sem = bufs[b]
                base = e_base + s * tile_e + i * _K
                pltpu.make_async_copy(
                    dst_hbm.at[pl.ds(base, _K)], di, sem).wait()
                pltpu.make_async_copy(
                    mt_hbm.at[pl.ds(base, _K), pl.ds(c0, 128)], rows,
                    sem).wait()
                pltpu.sync_copy(rows, acc.at[di], add=True)

            issue(0, 0)

            def body(j, _):
                i = 2 * j
                issue(1, i + 1)
                proc(0, i)

                @pl.when(i + 2 < n_chunks)
                def _():
                    issue(0, i + 2)

                proc(1, i + 1)
                return ()

            lax.fori_loop(0, n_chunks // 2, body, ())
            if n_chunks % 2:
                proc(0, n_chunks - 1)
            plsc.subcore_barrier()

        def dump(out_idx):
            for k in range(stripe // CH):
                pltpu.sync_copy(acc.at[pl.ds(s * stripe + k * CH, CH)], zb_v)
                pltpu.sync_copy(
                    zb_v, out_hbm.at[out_idx, pl.ds(s * stripe + k * CH, CH)])

        zero_acc()

        @pl.when(c == 0)
        def _():
            scan(0, 0, E // 16)
            dump(0)

        @pl.when(c == 1)
        def _():
            scan(128, 0, E // 16)
            dump(1)

        plsc.subcore_barrier()
        zero_acc()

        @pl.when(c == 0)
        def _():
            scan(256, 0, E // 32)
            dump(2)

        @pl.when(c == 1)
        def _():
            scan(256, E // 2, E // 32)
            dump(3)

    return scatter_kernel(mt, dst)


def _edge_tile_body(H, zs_ref, d_ref, e_ref, w2_ref, b2_ref, w1c_ref,
                    w1e_ref, cw_ref, cb_ref, eow_ref, eob_ref,
                    mt_ref, en_ref):
    TE = zs_ref.shape[0]
    z0 = zs_ref[...]
    dx = d_ref[:, 0:1]
    dy = d_ref[:, 1:2]
    d2 = dx * dx + dy * dy
    e = e_ref[...]
    z1 = (z0 + d2 * w1c_ref[...]
          + jnp.dot(e, w1e_ref[...], preferred_element_type=jnp.float32))
    m1 = _leaky(z1)
    m = _leaky(jnp.dot(m1, w2_ref[...], preferred_element_type=jnp.float32)
               + b2_ref[...])
    mt_ref[:, :H] = m
    coef = jnp.dot(m, cw_ref[...], preferred_element_type=jnp.float32) + cb_ref[...]
    s = coef / (jnp.sqrt(d2) + 1.0)
    col = lax.broadcasted_iota(jnp.int32, (TE, 128), 1)
    t128 = jnp.where(col == 0, dx * s,
                     jnp.where(col == 1, dy * s,
                               jnp.where(col == 2, 1.0, 0.0)))
    mt_ref[:, H:H + 128] = t128
    en_ref[...] = (jnp.dot(m, eow_ref[...], preferred_element_type=jnp.float32)
                   + eob_ref[...])


def _edge_mlp(Zs, diff, e, p, H, EF, TE):
    E = Zs.shape[0]
    grid = E // TE
    w2 = p['edge_w2']
    b2 = p['edge_b2'].reshape(1, H)
    w1c = p['edge_w1'][2 * H].reshape(1, H)
    w1e = p['edge_w1'][2 * H + 1:]
    cw = p['coord_w']
    cb = p['coord_b'].reshape(1, 1)
    eow = p['edge_out_w']
    eob = p['edge_out_b'].reshape(1, EF)
    full = lambda shape: pl.BlockSpec(shape, lambda i: (0, 0))
    return pl.pallas_call(
        functools.partial(_edge_tile_body, H),
        grid=(grid,),
        in_specs=[
            pl.BlockSpec((TE, H), lambda i: (i, 0)),
            pl.BlockSpec((TE, 2), lambda i: (i, 0)),
            pl.BlockSpec((TE, EF), lambda i: (i, 0)),
            full(w2.shape), full(b2.shape), full(w1c.shape), full(w1e.shape),
            full(cw.shape), full(cb.shape), full(eow.shape), full(eob.shape),
        ],
        out_specs=[
            pl.BlockSpec((TE, H + 128), lambda i: (i, 0)),
            pl.BlockSpec((TE, EF), lambda i: (i, 0)),
        ],
        out_shape=[
            jax.ShapeDtypeStruct((E, H + 128), jnp.float32),
            jax.ShapeDtypeStruct((E, EF), jnp.float32),
        ],
    )(Zs, diff, e, w2, b2, w1c, w1e, cw, cb, eow, eob)


def _node_a_body(h_ref, a0_ref, a1_ref, a2_ref, a3_ref, x_ref,
                 w1a_ref, w1bl_ref, w1bh_ref, nb1_ref, nw2_ref, nb2_ref,
                 hn2_ref, xn_ref, ps_ref, pq_ref):
    a2 = a2_ref[0]
    a3 = a3_ref[0]
    deg = jnp.maximum(a2[:, 2:3] + a3[:, 2:3], 1.0)
    invd = 1.0 / deg
    xn_ref[...] = x_ref[...] + (a2[:, 0:2] + a3[:, 0:2]) * invd
    mlow = a0_ref[0] * invd
    mhigh = a1_ref[0] * invd
    hn = _leaky(jnp.dot(h_ref[...], w1a_ref[...],
                        preferred_element_type=jnp.float32)
                + jnp.dot(mlow, w1bl_ref[...],
                          preferred_element_type=jnp.float32)
                + jnp.dot(mhigh, w1bh_ref[...],
                          preferred_element_type=jnp.float32)
                + nb1_ref[...])
    hn2 = (jnp.dot(hn, nw2_ref[...], preferred_element_type=jnp.float32)
           + nb2_ref[...])
    hn2_ref[...] = hn2
    ps_ref[...] = jnp.sum(hn2, axis=0, keepdims=True)[None]
    pq_ref[...] = jnp.sum(hn2 * hn2, axis=0, keepdims=True)[None]


def _node_a(h, agg, x, p, H, TN):
    n = h.shape[0]
    grid = n // TN
    w1a = p['node_w1'][:H]
    w1bl = p['node_w1'][H:H + 128]
    w1bh = p['node_w1'][H + 128:]
    nb1 = p['node_b1'].reshape(1, H)
    nw2 = p['node_w2']
    nb2 = p['node_b2'].reshape(1, H)
    full = lambda shape: pl.BlockSpec(shape, lambda i: (0, 0))
    row = lambda: pl.BlockSpec((TN, 128), lambda i: (i, 0))
    return pl.pallas_call(
        _node_a_body,
        grid=(grid,),
        in_specs=[
            pl.BlockSpec((TN, H), lambda i: (i, 0)),
            pl.BlockSpec((1, TN, 128), lambda i: (0, i, 0)),
            pl.BlockSpec((1, TN, 128), lambda i: (1, i, 0)),
            pl.BlockSpec((1, TN, 128), lambda i: (2, i, 0)),
            pl.BlockSpec((1, TN, 128), lambda i: (3, i, 0)),
            pl.BlockSpec((TN, 2), lambda i: (i, 0)),
            full(w1a.shape), full(w1bl.shape), full(w1bh.shape),
            full(nb1.shape), full(nw2.shape), full(nb2.shape),
        ],
        out_specs=[
            pl.BlockSpec((TN, H), lambda i: (i, 0)),
            pl.BlockSpec((TN, 2), lambda i: (i, 0)),
            pl.BlockSpec((1, 1, H), lambda i: (i, 0, 0)),
            pl.BlockSpec((1, 1, H), lambda i: (i, 0, 0)),
        ],
        out_shape=[
            jax.ShapeDtypeStruct((n, H), jnp.float32),
            jax.ShapeDtypeStruct((n, 2), jnp.float32),
            jax.ShapeDtypeStruct((grid, 1, H), jnp.float32),
            jax.ShapeDtypeStruct((grid, 1, H), jnp.float32),
        ],
    )(h, agg, agg, agg, agg, x,
      w1a, w1bl, w1bh, nb1, nw2, nb2)


def _node_b_body(hn2_ref, mu_ref, iv_ref, g_ref, b_ref, wp_ref, wq_ref,
                 bp_ref, h_ref, p_ref, q_ref):
    hx = _leaky((hn2_ref[...] - mu_ref[...]) * iv_ref[...] * g_ref[...]
                + b_ref[...])
    h_ref[...] = hx
    p_ref[...] = (jnp.dot(hx, wp_ref[...], preferred_element_type=jnp.float32)
                  + bp_ref[...])
    q_ref[...] = jnp.dot(hx, wq_ref[...], preferred_element_type=jnp.float32)


def _node_b(hn2, mu, iv, p, p_next, H, TN):
    n = hn2.shape[0]
    grid = n // TN
    g = p['bn_gamma'].reshape(1, H)
    b = p['bn_beta'].reshape(1, H)
    wp = p_next['edge_w1'][:H]
    wq = p_next['edge_w1'][H:2 * H]
    bp = p_next['edge_b1'].reshape(1, H)
    full = lambda shape: pl.BlockSpec(shape, lambda i: (0, 0))
    return pl.pallas_call(
        _node_b_body,
        grid=(grid,),
        in_specs=[
            pl.BlockSpec((TN, H), lambda i: (i, 0)),
            full((1, H)), full((1, H)), full((1, H)), full((1, H)),
            full(wp.shape), full(wq.shape), full((1, H)),
        ],
        out_specs=[
            pl.BlockSpec((TN, H), lambda i: (i, 0)),
            pl.BlockSpec((TN, H), lambda i: (i, 0)),
            pl.BlockSpec((TN, H), lambda i: (i, 0)),
        ],
        out_shape=[
            jax.ShapeDtypeStruct((n, H), jnp.float32),
            jax.ShapeDtypeStruct((n, H), jnp.float32),
            jax.ShapeDtypeStruct((n, H), jnp.float32),
        ],
    )(hn2, mu.reshape(1, H), iv.reshape(1, H), g, b, wp, wq, bp)


def _node_b_last_body(hn2_ref, mu_ref, iv_ref, g_ref, b_ref, rw_ref, rb_ref,
                      o_ref):
    hx = _leaky((hn2_ref[...] - mu_ref[...]) * iv_ref[...] * g_ref[...]
                + b_ref[...])
    o_ref[...] = (jnp.dot(hx, rw_ref[...], preferred_element_type=jnp.float32)
                  + rb_ref[...])


def _node_b_last(hn2, mu, iv, p, rw, rb, H, TN):
    n = hn2.shape[0]
    grid = n // TN
    g = p['bn_gamma'].reshape(1, H)
    b = p['bn_beta'].reshape(1, H)
    full = lambda shape: pl.BlockSpec(shape, lambda i: (0, 0))
    return pl.pallas_call(
        _node_b_last_body,
        grid=(grid,),
        in_specs=[
            pl.BlockSpec((TN, H), lambda i: (i, 0)),
            full((1, H)), full((1, H)), full((1, H)), full((1, H)),
            full(rw.shape), full((1, rb.shape[0])),
        ],
        out_specs=pl.BlockSpec((TN, rw.shape[1]), lambda i: (i, 0)),
        out_shape=jax.ShapeDtypeStruct((n, rw.shape[1]), jnp.float32),
    )(hn2, mu.reshape(1, H), iv.reshape(1, H), g, b, rw,
      rb.reshape(1, -1))


def kernel(coords, node_feat, edge_attr, params, edge_index, batch_index, t):
    n, d = node_feat.shape
    E = edge_attr.shape[0]
    EF = edge_attr.shape[1]
    H = 2 * d
    TE = 2560 if E % 2560 == 0 else E
    TN = 2000 if n % 2000 == 0 else n
    t_idx = jnp.full((n,), t, dtype=jnp.int32)
    t_embed = jnp.take(params['time_embed'], t_idx, axis=0)
    h = jnp.concatenate([node_feat, t_embed], axis=-1)
    x = coords
    e = edge_attr
    src = edge_index[0]
    dst = edge_index[1]
    layers = params['layers']
    P = h @ layers[0]['edge_w1'][:H] + layers[0]['edge_b1']
    Q = h @ layers[0]['edge_w1'][H:2 * H]
    pred = None
    for li, p in enumerate(layers):
        Zs, dflat = _gather_pq(P, Q, x.reshape(-1), src, dst)
        diff = dflat.reshape(-1, 2)
        mt, e = _edge_mlp(Zs, diff, e, p, H, EF, TE)
        agg = _scatter_mt(mt, dst)
        hn2, x, ps, pq = _node_a(h, agg, x, p, H, TN)
        mu = jnp.sum(ps, axis=(0, 1)) / n
        var = jnp.sum(pq, axis=(0, 1)) / n - mu * mu
        iv = lax.rsqrt(var + 1e-5)
        if li + 1 < len(layers):
            h, P, Q = _node_b(hn2, mu, iv, p, layers[li + 1], H, TN)
        else:
            pred = _node_b_last(hn2, mu, iv, p, params['readout_w'],
                                params['readout_b'], H, TN)
    return (x, pred)


# speedup vs baseline: 3.4670x; 1.0020x over previous
import functools

import jax
import jax.numpy as jnp
from jax import lax
from jax.experimental import pallas as pl
from jax.experimental.pallas import tpu as pltpu
from jax.experimental.pallas import tpu_sc as plsc

_SLOPE = 0.01

_NW = 32
_K = 80


def _leaky(v):
    return jnp.where(v > 0, v, _SLOPE * v)


def _gather_pq(Px, Qx, x_flat, src, dst):
    N, Wp = Px.shape
    E = src.shape[0]
    per_w = E // _NW
    steps = per_w // _K
    mesh = plsc.VectorSubcoreMesh(core_axis_name="c", subcore_axis_name="s")

    @functools.partial(
        pl.kernel, mesh=mesh,
        out_type=[jax.ShapeDtypeStruct((E, Wp), jnp.float32),
                  jax.ShapeDtypeStruct((2 * E,), jnp.float32)],
        scratch_types=[
            pltpu.VMEM((per_w,), jnp.int32),
            pltpu.VMEM((per_w,), jnp.int32),
            pltpu.VMEM((_K, Wp), jnp.float32),
            pltpu.VMEM((_K, Wp), jnp.float32),
            pltpu.VMEM((_K, Wp), jnp.float32),
            pltpu.VMEM((_K, Wp), jnp.float32),
            pltpu.VMEM((2 * _K,), jnp.float32),
            pltpu.VMEM((2 * _K,), jnp.float32),
            pltpu.VMEM((x_flat.shape[0],), jnp.float32),
            pltpu.SemaphoreType.DMA,
            pltpu.SemaphoreType.DMA,
        ],
        compiler_params=pltpu.CompilerParams(needs_layout_passes=False),
    )
    def gather_kernel(px_hbm, qx_hbm, xf_hbm, src_hbm, dst_hbm,
                      zs_hbm, df_hbm,
                      di_all, si_all, rp0, rq0, rp1, rq1, db0, db1, xv,
                      gsem0, gsem1):
        wid = lax.axis_index("s") * 2 + lax.axis_index("c")
        w_base = wid * per_w
        pltpu.sync_copy(xf_hbm, xv)
        pltpu.sync_copy(dst_hbm.at[pl.ds(w_base, per_w)], di_all)
        pltpu.sync_copy(src_hbm.at[pl.ds(w_base, per_w)], si_all)
        lane = lax.iota(jnp.int32, 16)
        bufs = ((rp0, rq0, db0, gsem0), (rp1, rq1, db1, gsem1))

        def issue(b, c):
            rp, rq, _, gsem = bufs[b]
            off = c * _K
            pltpu.async_copy(px_hbm.at[di_all.at[pl.ds(off, _K)]], rp, gsem)
            pltpu.async_copy(qx_hbm.at[si_all.at[pl.ds(off, _K)]], rq, gsem)

        def proc(b, c):
            rp, rq, db, gsem = bufs[b]
            off = c * _K
            pltpu.make_async_copy(
                px_hbm.at[di_all.at[pl.ds(off, _K)]], rp, gsem).wait()
            pltpu.make_async_copy(
                qx_hbm.at[si_all.at[pl.ds(off, _K)]], rq, gsem).wait()

            def addrow(r, _):
                for t in range(Wp // 16):
                    sl = pl.ds(16 * t, 16)
                    rp[r, sl] = rp[r, sl] + rq[r, sl]
                return ()

            lax.fori_loop(0, _K, addrow, ())
            for j in range(_K // 16):
                d16 = di_all[pl.ds(off + 16 * j, 16)]
                s16 = si_all[pl.ds(off + 16 * j, 16)]
                dxv = (plsc.load_gather(xv, [d16 * 2])
                       - plsc.load_gather(xv, [s16 * 2]))
                dyv = (plsc.load_gather(xv, [d16 * 2 + 1])
                       - plsc.load_gather(xv, [s16 * 2 + 1]))
                pos = 32 * j + 2 * lane
                plsc.store_scatter(db, [pos], dxv)
                plsc.store_scatter(db, [pos + 1], dyv)
            base = w_base + off
            pltpu.sync_copy(rp, zs_hbm.at[pl.ds(base, _K)])
            pltpu.sync_copy(db, df_hbm.at[pl.ds(2 * base, 2 * _K)])

        issue(0, 0)

        def body(j, _):
            c = 2 * j
            issue(1, c + 1)
            proc(0, c)

            @pl.when(c + 2 < steps)
            def _():
                issue(0, c + 2)

            proc(1, c + 1)
            return ()

        lax.fori_loop(0, steps // 2, body, ())
        proc(0, steps - 1)

    return gather_kernel(Px, Qx, x_flat, src, dst)


def _scatter_mt(mt, dst):
    E, W = mt.shape
    n_rows = 10240
    stripe = n_rows // 16
    CH = 32
    mesh = plsc.VectorSubcoreMesh(core_axis_name="c", subcore_axis_name="s")

    @functools.partial(
        pl.kernel, mesh=mesh,
        out_type=jax.ShapeDtypeStruct((4, n_rows, 128), jnp.float32),
        scratch_types=[
            pltpu.VMEM((_K,), jnp.int32),
            pltpu.VMEM((_K,), jnp.int32),
            pltpu.VMEM((_K, 128), jnp.float32),
            pltpu.VMEM((_K, 128), jnp.float32),
            pltpu.VMEM((CH, 128), jnp.float32),
            pltpu.VMEM_SHARED((n_rows, 128), jnp.float32),
            pltpu.SemaphoreType.DMA,
            pltpu.SemaphoreType.DMA,
        ],
        compiler_params=pltpu.CompilerParams(needs_layout_passes=False),
    )
    def scatter_kernel(mt_hbm, dst_hbm, out_hbm,
                       di0, di1, rows0, rows1, zb_v, acc, sem0, sem1):
        c = lax.axis_index("c")
        s = lax.axis_index("s")
        bufs = ((di0, rows0, sem0), (di1, rows1, sem1))

        def zero_acc():
            def zrow(r, _):
                for j in range(128 // 16):
                    zb_v[r, pl.ds(16 * j, 16)] = jnp.zeros((16,), jnp.float32)
                return ()
            lax.fori_loop(0, CH, zrow, ())
            for k in range(stripe // CH):
                pltpu.sync_copy(zb_v, acc.at[pl.ds(s * stripe + k * CH, CH)])
            plsc.subcore_barrier()

        def scan(c0, e_base, tile_e):
            n_chunks = tile_e // _K

            def issue(b, i):
                di, rows, sem = bufs[b]
                base = e_base + s * tile_e + i * _K
                pltpu.async_copy(dst_hbm.at[pl.ds(base, _K)], di, sem)
                pltpu.async_copy(
                    mt_hbm.at[pl.ds(base, _K), pl.ds(c0, 128)], rows, sem)

            def proc(b, i):
                di, rows, sem = bufs[b]
                base = e_base + s * tile_e + i * _K
                pltpu.make_async_copy(
                    dst_hbm.at[pl.ds(base, _K)], di, sem).wait()
                pltpu.make_async_copy(
                    mt_hbm.at[pl.ds(base, _K), pl.ds(c0, 128)], rows,
                    sem).wait()
                pltpu.sync_copy(rows, acc.at[di], add=True)

            issue(0, 0)

            def body(j, _):
                i = 2 * j
                issue(1, i + 1)
                proc(0, i)

                @pl.when(i + 2 < n_chunks)
                def _():
                    issue(0, i + 2)

                proc(1, i + 1)
                return ()

            lax.fori_loop(0, n_chunks // 2, body, ())
            if n_chunks % 2:
                proc(0, n_chunks - 1)
            plsc.subcore_barrier()

        def dump(out_idx):
            for k in range(stripe // CH):
                pltpu.sync_copy(acc.at[pl.ds(s * stripe + k * CH, CH)], zb_v)
                pltpu.sync_copy(
                    zb_v, out_hbm.at[out_idx, pl.ds(s * stripe + k * CH, CH)])

        zero_acc()

        @pl.when(c == 0)
        def _():
            scan(0, 0, E // 16)
            dump(0)

        @pl.when(c == 1)
        def _():
            scan(128, 0, E // 16)
            dump(1)

        plsc.subcore_barrier()
        zero_acc()

        @pl.when(c == 0)
        def _():
            scan(256, 0, E // 32)
            dump(2)

        @pl.when(c == 1)
        def _():
            scan(256, E // 2, E // 32)
            dump(3)

    return scatter_kernel(mt, dst)


def _edge_tile_body(H, zs_ref, d_ref, e_ref, w2_ref, b2_ref, w1c_ref,
                    w1e_ref, cw_ref, cb_ref, eow_ref, eob_ref,
                    mt_ref, en_ref):
    TE = zs_ref.shape[0]
    z0 = zs_ref[...]
    dx = d_ref[:, 0:1]
    dy = d_ref[:, 1:2]
    d2 = dx * dx + dy * dy
    e = e_ref[...]
    z1 = (z0 + d2 * w1c_ref[...]
          + jnp.dot(e, w1e_ref[...], preferred_element_type=jnp.float32))
    m1 = _leaky(z1)
    m = _leaky(jnp.dot(m1.astype(jnp.bfloat16),
                       w2_ref[...].astype(jnp.bfloat16),
                       preferred_element_type=jnp.float32)
               + b2_ref[...])
    mt_ref[:, :H] = m
    coef = jnp.dot(m, cw_ref[...], preferred_element_type=jnp.float32) + cb_ref[...]
    s = coef / (jnp.sqrt(d2) + 1.0)
    col = lax.broadcasted_iota(jnp.int32, (TE, 128), 1)
    t128 = jnp.where(col == 0, dx * s,
                     jnp.where(col == 1, dy * s,
                               jnp.where(col == 2, 1.0, 0.0)))
    mt_ref[:, H:H + 128] = t128
    en_ref[...] = (jnp.dot(m, eow_ref[...], preferred_element_type=jnp.float32)
                   + eob_ref[...])


def _edge_mlp(Zs, diff, e, p, H, EF, TE):
    E = Zs.shape[0]
    grid = E // TE
    w2 = p['edge_w2']
    b2 = p['edge_b2'].reshape(1, H)
    w1c = p['edge_w1'][2 * H].reshape(1, H)
    w1e = p['edge_w1'][2 * H + 1:]
    cw = p['coord_w']
    cb = p['coord_b'].reshape(1, 1)
    eow = p['edge_out_w']
    eob = p['edge_out_b'].reshape(1, EF)
    full = lambda shape: pl.BlockSpec(shape, lambda i: (0, 0))
    return pl.pallas_call(
        functools.partial(_edge_tile_body, H),
        grid=(grid,),
        in_specs=[
            pl.BlockSpec((TE, H), lambda i: (i, 0)),
            pl.BlockSpec((TE, 2), lambda i: (i, 0)),
            pl.BlockSpec((TE, EF), lambda i: (i, 0)),
            full(w2.shape), full(b2.shape), full(w1c.shape), full(w1e.shape),
            full(cw.shape), full(cb.shape), full(eow.shape), full(eob.shape),
        ],
        out_specs=[
            pl.BlockSpec((TE, H + 128), lambda i: (i, 0)),
            pl.BlockSpec((TE, EF), lambda i: (i, 0)),
        ],
        out_shape=[
            jax.ShapeDtypeStruct((E, H + 128), jnp.float32),
            jax.ShapeDtypeStruct((E, EF), jnp.float32),
        ],
    )(Zs, diff, e, w2, b2, w1c, w1e, cw, cb, eow, eob)


def _node_a_body(h_ref, a0_ref, a1_ref, a2_ref, a3_ref, x_ref,
                 w1a_ref, w1bl_ref, w1bh_ref, nb1_ref, nw2_ref, nb2_ref,
                 hn2_ref, xn_ref, ps_ref, pq_ref):
    a2 = a2_ref[0]
    a3 = a3_ref[0]
    deg = jnp.maximum(a2[:, 2:3] + a3[:, 2:3], 1.0)
    invd = 1.0 / deg
    xn_ref[...] = x_ref[...] + (a2[:, 0:2] + a3[:, 0:2]) * invd
    mlow = a0_ref[0] * invd
    mhigh = a1_ref[0] * invd
    hn = _leaky(jnp.dot(h_ref[...], w1a_ref[...],
                        preferred_element_type=jnp.float32)
                + jnp.dot(mlow, w1bl_ref[...],
                          preferred_element_type=jnp.float32)
                + jnp.dot(mhigh, w1bh_ref[...],
                          preferred_element_type=jnp.float32)
                + nb1_ref[...])
    hn2 = (jnp.dot(hn, nw2_ref[...], preferred_element_type=jnp.float32)
           + nb2_ref[...])
    hn2_ref[...] = hn2
    ps_ref[...] = jnp.sum(hn2, axis=0, keepdims=True)[None]
    pq_ref[...] = jnp.sum(hn2 * hn2, axis=0, keepdims=True)[None]


def _node_a(h, agg, x, p, H, TN):
    n = h.shape[0]
    grid = n // TN
    w1a = p['node_w1'][:H]
    w1bl = p['node_w1'][H:H + 128]
    w1bh = p['node_w1'][H + 128:]
    nb1 = p['node_b1'].reshape(1, H)
    nw2 = p['node_w2']
    nb2 = p['node_b2'].reshape(1, H)
    full = lambda shape: pl.BlockSpec(shape, lambda i: (0, 0))
    row = lambda: pl.BlockSpec((TN, 128), lambda i: (i, 0))
    return pl.pallas_call(
        _node_a_body,
        grid=(grid,),
        in_specs=[
            pl.BlockSpec((TN, H), lambda i: (i, 0)),
            pl.BlockSpec((1, TN, 128), lambda i: (0, i, 0)),
            pl.BlockSpec((1, TN, 128), lambda i: (1, i, 0)),
            pl.BlockSpec((1, TN, 128), lambda i: (2, i, 0)),
            pl.BlockSpec((1, TN, 128), lambda i: (3, i, 0)),
            pl.BlockSpec((TN, 2), lambda i: (i, 0)),
            full(w1a.shape), full(w1bl.shape), full(w1bh.shape),
            full(nb1.shape), full(nw2.shape), full(nb2.shape),
        ],
        out_specs=[
            pl.BlockSpec((TN, H), lambda i: (i, 0)),
            pl.BlockSpec((TN, 2), lambda i: (i, 0)),
            pl.BlockSpec((1, 1, H), lambda i: (i, 0, 0)),
            pl.BlockSpec((1, 1, H), lambda i: (i, 0, 0)),
        ],
        out_shape=[
            jax.ShapeDtypeStruct((n, H), jnp.float32),
            jax.ShapeDtypeStruct((n, 2), jnp.float32),
            jax.ShapeDtypeStruct((grid, 1, H), jnp.float32),
            jax.ShapeDtypeStruct((grid, 1, H), jnp.float32),
        ],
    )(h, agg, agg, agg, agg, x,
      w1a, w1bl, w1bh, nb1, nw2, nb2)


def _node_b_body(hn2_ref, mu_ref, iv_ref, g_ref, b_ref, wp_ref, wq_ref,
                 bp_ref, h_ref, p_ref, q_ref):
    hx = _leaky((hn2_ref[...] - mu_ref[...]) * iv_ref[...] * g_ref[...]
                + b_ref[...])
    h_ref[...] = hx
    p_ref[...] = (jnp.dot(hx, wp_ref[...], preferred_element_type=jnp.float32)
                  + bp_ref[...])
    q_ref[...] = jnp.dot(hx, wq_ref[...], preferred_element_type=jnp.float32)


def _node_b(hn2, mu, iv, p, p_next, H, TN):
    n = hn2.shape[0]
    grid = n // TN
    g = p['bn_gamma'].reshape(1, H)
    b = p['bn_beta'].reshape(1, H)
    wp = p_next['edge_w1'][:H]
    wq = p_next['edge_w1'][H:2 * H]
    bp = p_next['edge_b1'].reshape(1, H)
    full = lambda shape: pl.BlockSpec(shape, lambda i: (0, 0))
    return pl.pallas_call(
        _node_b_body,
        grid=(grid,),
        in_specs=[
            pl.BlockSpec((TN, H), lambda i: (i, 0)),
            full((1, H)), full((1, H)), full((1, H)), full((1, H)),
            full(wp.shape), full(wq.shape), full((1, H)),
        ],
        out_specs=[
            pl.BlockSpec((TN, H), lambda i: (i, 0)),
            pl.BlockSpec((TN, H), lambda i: (i, 0)),
            pl.BlockSpec((TN, H), lambda i: (i, 0)),
        ],
        out_shape=[
            jax.ShapeDtypeStruct((n, H), jnp.float32),
            jax.ShapeDtypeStruct((n, H), jnp.float32),
            jax.ShapeDtypeStruct((n, H), jnp.float32),
        ],
    )(hn2, mu.reshape(1, H), iv.reshape(1, H), g, b, wp, wq, bp)


def _node_b_last_body(hn2_ref, mu_ref, iv_ref, g_ref, b_ref, rw_ref, rb_ref,
                      o_ref):
    hx = _leaky((hn2_ref[...] - mu_ref[...]) * iv_ref[...] * g_ref[...]
                + b_ref[...])
    o_ref[...] = (jnp.dot(hx, rw_ref[...], preferred_element_type=jnp.float32)
                  + rb_ref[...])


def _node_b_last(hn2, mu, iv, p, rw, rb, H, TN):
    n = hn2.shape[0]
    grid = n // TN
    g = p['bn_gamma'].reshape(1, H)
    b = p['bn_beta'].reshape(1, H)
    full = lambda shape: pl.BlockSpec(shape, lambda i: (0, 0))
    return pl.pallas_call(
        _node_b_last_body,
        grid=(grid,),
        in_specs=[
            pl.BlockSpec((TN, H), lambda i: (i, 0)),
            full((1, H)), full((1, H)), full((1, H)), full((1, H)),
            full(rw.shape), full((1, rb.shape[0])),
        ],
        out_specs=pl.BlockSpec((TN, rw.shape[1]), lambda i: (i, 0)),
        out_shape=jax.ShapeDtypeStruct((n, rw.shape[1]), jnp.float32),
    )(hn2, mu.reshape(1, H), iv.reshape(1, H), g, b, rw,
      rb.reshape(1, -1))


def kernel(coords, node_feat, edge_attr, params, edge_index, batch_index, t):
    n, d = node_feat.shape
    E = edge_attr.shape[0]
    EF = edge_attr.shape[1]
    H = 2 * d
    TE = 2560 if E % 2560 == 0 else E
    TN = 2000 if n % 2000 == 0 else n
    t_idx = jnp.full((n,), t, dtype=jnp.int32)
    t_embed = jnp.take(params['time_embed'], t_idx, axis=0)
    h = jnp.concatenate([node_feat, t_embed], axis=-1)
    x = coords
    e = edge_attr
    src = edge_index[0]
    dst = edge_index[1]
    layers = params['layers']
    P = h @ layers[0]['edge_w1'][:H] + layers[0]['edge_b1']
    Q = h @ layers[0]['edge_w1'][H:2 * H]
    pred = None
    for li, p in enumerate(layers):
        Zs, dflat = _gather_pq(P, Q, x.reshape(-1), src, dst)
        diff = dflat.reshape(-1, 2)
        mt, e = _edge_mlp(Zs, diff, e, p, H, EF, TE)
        agg = _scatter_mt(mt, dst)
        hn2, x, ps, pq = _node_a(h, agg, x, p, H, TN)
        mu = jnp.sum(ps, axis=(0, 1)) / n
        var = jnp.sum(pq, axis=(0, 1)) / n - mu * mu
        iv = lax.rsqrt(var + 1e-5)
        if li + 1 < len(layers):
            h, P, Q = _node_b(hn2, mu, iv, p, layers[li + 1], H, TN)
        else:
            pred = _node_b_last(hn2, mu, iv, p, params['readout_w'],
                                params['readout_b'], H, TN)
    return (x, pred)
